# baseline (speedup 1.0000x reference)
import jax
import jax.numpy as jnp
from jax import lax
from jax.experimental import pallas as pl
from jax.experimental.pallas import tpu as pltpu
from jax.experimental.pallas import tpu_sc as plsc

_N_IDS = 8192
_N_BINS = 256
_N_WORKERS = 8
_IDS_PER_WORKER = _N_IDS // _N_WORKERS
_L = 16


def _hist_body(ids_hbm, out_hbm, ids_v, counts_v):
    wid = lax.axis_index("s")
    pltpu.sync_copy(ids_hbm.at[pl.ds(wid * _IDS_PER_WORKER, _IDS_PER_WORKER)],
                    ids_v)
    zeros = jnp.zeros((_L,), jnp.float32)
    for j in range(_N_BINS // _L):
        counts_v[pl.ds(j * _L, _L)] = zeros
    ones = jnp.ones((_L,), jnp.float32)
    for j in range(_IDS_PER_WORKER // _L):
        plsc.addupdate_scatter(counts_v, [ids_v[pl.ds(j * _L, _L)]], ones)
    pltpu.sync_copy(counts_v, out_hbm.at[wid])


_hist = pl.kernel(
    _hist_body,
    mesh=plsc.VectorSubcoreMesh(core_axis_name="c", subcore_axis_name="s",
                                num_cores=1, num_subcores=8),
    out_type=jax.ShapeDtypeStruct((_N_WORKERS, _N_BINS), jnp.float32),
    scratch_types=[
        pltpu.VMEM((_IDS_PER_WORKER,), jnp.int32),
        pltpu.VMEM((_N_BINS,), jnp.float32),
    ],
    compiler_params=pltpu.CompilerParams(needs_layout_passes=False,
                                         disable_bounds_checks=True),
)


def _mlp_body(pc_ref, emb_ref, w1_ref, b1_ref, w2_ref, b2_ref, out_ref):
    counts = jnp.sum(pc_ref[...], axis=0, keepdims=True)
    e = lax.dot_general(counts, emb_ref[...],
                        (((1,), (0,)), ((), ())),
                        precision=lax.Precision.HIGHEST,
                        preferred_element_type=jnp.float32) * (1.0 / _N_IDS)
    h = lax.dot_general(e, w1_ref[...],
                        (((1,), (1,)), ((), ())),
                        preferred_element_type=jnp.float32) + b1_ref[...]
    h = jnp.maximum(h, 0.0)
    v = lax.dot_general(h, w2_ref[...],
                        (((1,), (1,)), ((), ())),
                        preferred_element_type=jnp.float32) + b2_ref[...]
    out_ref[...] = v


def kernel(ids, emb, W1, b1, W2, b2):
    ids32 = ids.astype(jnp.int32)
    partial_counts = _hist(ids32)
    out = pl.pallas_call(
        _mlp_body,
        out_shape=jax.ShapeDtypeStruct((1, 8), jnp.float32),
    )(partial_counts, emb, W1, b1.reshape(1, 64), W2, b2.reshape(1, 8))
    return out[0]


# speedup vs baseline: 1.0121x; 1.0121x over previous
import jax
import jax.numpy as jnp
from jax import lax
from jax.experimental import pallas as pl
from jax.experimental.pallas import tpu as pltpu
from jax.experimental.pallas import tpu_sc as plsc

_N_IDS = 8192
_N_BINS = 256
_N_WORKERS = 16
_IDS_PER_WORKER = _N_IDS // _N_WORKERS
_L = 16


def _hist_body(ids_hbm, out_hbm, ids_v, counts_v):
    wid = lax.axis_index("s")
    pltpu.sync_copy(ids_hbm.at[pl.ds(wid * _IDS_PER_WORKER, _IDS_PER_WORKER)],
                    ids_v)
    zeros = jnp.zeros((_L,), jnp.float32)
    for j in range(_N_BINS // _L):
        counts_v[pl.ds(j * _L, _L)] = zeros
    ones = jnp.ones((_L,), jnp.float32)
    for j in range(_IDS_PER_WORKER // _L):
        plsc.addupdate_scatter(counts_v, [ids_v[pl.ds(j * _L, _L)]], ones)
    pltpu.sync_copy(counts_v, out_hbm.at[wid])


_hist = pl.kernel(
    _hist_body,
    mesh=plsc.VectorSubcoreMesh(core_axis_name="c", subcore_axis_name="s",
                                num_cores=1),
    out_type=jax.ShapeDtypeStruct((_N_WORKERS, _N_BINS), jnp.float32),
    scratch_types=[
        pltpu.VMEM((_IDS_PER_WORKER,), jnp.int32),
        pltpu.VMEM((_N_BINS,), jnp.float32),
    ],
    compiler_params=pltpu.CompilerParams(needs_layout_passes=False,
                                         disable_bounds_checks=True),
)


def _mlp_body(pc_ref, emb_ref, w1_ref, b1_ref, w2_ref, b2_ref, out_ref):
    counts = jnp.sum(pc_ref[...], axis=0, keepdims=True)
    e = lax.dot_general(counts, emb_ref[...],
                        (((1,), (0,)), ((), ())),
                        precision=lax.Precision.HIGHEST,
                        preferred_element_type=jnp.float32) * (1.0 / _N_IDS)
    h = lax.dot_general(e, w1_ref[...],
                        (((1,), (1,)), ((), ())),
                        preferred_element_type=jnp.float32) + b1_ref[...]
    h = jnp.maximum(h, 0.0)
    v = lax.dot_general(h, w2_ref[...],
                        (((1,), (1,)), ((), ())),
                        preferred_element_type=jnp.float32) + b2_ref[...]
    out_ref[...] = v


def kernel(ids, emb, W1, b1, W2, b2):
    ids32 = ids.astype(jnp.int32)
    partial_counts = _hist(ids32)
    out = pl.pallas_call(
        _mlp_body,
        out_shape=jax.ShapeDtypeStruct((1, 8), jnp.float32),
    )(partial_counts, emb, W1, b1.reshape(1, 64), W2, b2.reshape(1, 8))
    return out[0]
